# Initial kernel scaffold; baseline (speedup 1.0000x reference)
import jax
import jax.numpy as jnp
from jax.experimental import pallas as pl


def kernel(x, edge_index, batch, W1, b1, W2, b2, Wl, bl):
    raise NotImplementedError("write your pallas kernel here")


# speedup vs baseline: 12.9290x; 12.9290x over previous
import functools

import jax
import jax.numpy as jnp
from jax import lax
from jax.experimental import pallas as pl
from jax.experimental.pallas import tpu as pltpu
from jax.experimental.pallas import tpu_sc as plsc

NC = 2
NS = 16
NW = NC * NS

@functools.cache
def _mesh():
    return plsc.VectorSubcoreMesh(
        core_axis_name="c", subcore_axis_name="s", num_cores=NC, num_subcores=NS
    )

F32 = jnp.float32


def _fill(ref, rows, width, value):
    per_row = width // 16

    def body(j, _):
        r = j // per_row
        k = j % per_row
        ref[r, pl.ds(k * 16, 16)] = jnp.full((16,), value, F32)
        return _

    lax.fori_loop(0, rows * per_row, body, None)


def _make_deg(N, NP, E):
    EPT = E // NW
    K = 80
    assert E % NW == 0 and EPT % K == 0 and NP % (8 * NS) == 0
    NCH = EPT // K
    RPT = NP // NS
    ZR = 128
    assert RPT % ZR == 0

    @functools.partial(
        pl.kernel,
        out_type=jax.ShapeDtypeStruct((2 * NP, 16), F32),
        mesh=_mesh(),
        scratch_types=[
            pltpu.VMEM((1, K), jnp.int32),
            pltpu.VMEM((K, 16), F32),
            pltpu.VMEM((ZR, 16), F32),
            pltpu.VMEM_SHARED((NP, 16), F32),
        ],
    )
    def deg_kernel(dst_hbm, out_hbm, idx_v, ones_v, zb_v, acc):
        c = lax.axis_index("c")
        s = lax.axis_index("s")
        _fill(ones_v, K, 16, 1.0)
        _fill(zb_v, ZR, 16, 0.0)

        def fz(j, _):
            pltpu.sync_copy(zb_v, acc.at[pl.ds(s * RPT + j * ZR, ZR)])
            return _

        lax.fori_loop(0, RPT // ZR, fz, None)
        plsc.subcore_barrier()

        base = (c * NS + s) * EPT

        def fe(i, _):
            pltpu.sync_copy(dst_hbm.at[pl.ds(base + i * K, K)], idx_v.at[0])
            pltpu.sync_copy(ones_v, acc.at[idx_v.at[0]], add=True)
            return _

        lax.fori_loop(0, NCH, fe, None)
        plsc.subcore_barrier()
        pltpu.sync_copy(
            acc.at[pl.ds(s * RPT, RPT)],
            out_hbm.at[pl.ds(c * NP + s * RPT, RPT)],
        )

    return deg_kernel


def _make_prop(N, NP, E, H):
    EPT = E // NW
    K = 80
    assert EPT % K == 0 and NP % (8 * NS) == 0
    NCH = EPT // K
    RPT = NP // NS
    ZR = 128
    assert RPT % ZR == 0

    @functools.partial(
        pl.kernel,
        out_type=jax.ShapeDtypeStruct((2 * NP, H), F32),
        mesh=_mesh(),
        scratch_types=[
            pltpu.VMEM((1, K), jnp.int32),
            pltpu.VMEM((1, K), jnp.int32),
            pltpu.VMEM((K, H), F32),
            pltpu.VMEM((ZR, H), F32),
            pltpu.VMEM_SHARED((NP, H), F32),
            pltpu.SemaphoreType.DMA,
        ],
    )
    def prop_kernel(hs_hbm, src_hbm, dst_hbm, out_hbm, sidx, didx, rows_v,
                    zb_v, acc, sem):
        c = lax.axis_index("c")
        s = lax.axis_index("s")
        _fill(zb_v, ZR, H, 0.0)

        def fz(j, _):
            pltpu.sync_copy(zb_v, acc.at[pl.ds(s * RPT + j * ZR, ZR)])
            return _

        lax.fori_loop(0, RPT // ZR, fz, None)
        plsc.subcore_barrier()

        base = (c * NS + s) * EPT

        def fe(i, _):
            pltpu.sync_copy(src_hbm.at[pl.ds(base + i * K, K)], sidx.at[0])
            pltpu.sync_copy(dst_hbm.at[pl.ds(base + i * K, K)], didx.at[0])
            pltpu.async_copy(hs_hbm.at[sidx.at[0]], rows_v, sem).wait()
            pltpu.sync_copy(rows_v, acc.at[didx.at[0]], add=True)
            return _

        lax.fori_loop(0, NCH, fe, None)
        plsc.subcore_barrier()
        pltpu.sync_copy(
            acc.at[pl.ds(s * RPT, RPT)],
            out_hbm.at[pl.ds(c * NP + s * RPT, RPT)],
        )

    return prop_kernel


def _make_pool(N, H, G):
    K = 80
    NPT = -(-N // (NW * K)) * K
    assert NPT % K == 0

    @functools.partial(
        pl.kernel,
        out_type=(
            jax.ShapeDtypeStruct((2 * G, H), F32),
            jax.ShapeDtypeStruct((2 * G, 16), F32),
        ),
        mesh=_mesh(),
        scratch_types=[
            pltpu.VMEM((1, K), jnp.int32),
            pltpu.VMEM((K, 16), F32),
            pltpu.VMEM((K, H), F32),
            pltpu.VMEM((G, H), F32),
            pltpu.VMEM((G, 16), F32),
            pltpu.VMEM_SHARED((G, H), F32),
            pltpu.VMEM_SHARED((G, 16), F32),
        ],
    )
    def pool_kernel(h2_hbm, batch_hbm, outp_hbm, outc_hbm, idx_v, ones_v,
                    rows_v, zbp_v, zbc_v, accp, accc):
        c = lax.axis_index("c")
        s = lax.axis_index("s")
        _fill(ones_v, K, 16, 1.0)

        @pl.when(s == 0)
        def _():
            _fill(zbp_v, G, H, 0.0)
            _fill(zbc_v, G, 16, 0.0)
            pltpu.sync_copy(zbp_v, accp)
            pltpu.sync_copy(zbc_v, accc)

        plsc.subcore_barrier()

        wid = c * NS + s
        base = wid * NPT
        todo = jnp.maximum(jnp.minimum(NPT, N - base), 0)
        nch = todo // K

        def fn(j, _):
            b = base + j * K
            pltpu.sync_copy(batch_hbm.at[pl.ds(b, K)], idx_v.at[0])
            pltpu.sync_copy(h2_hbm.at[pl.ds(b, K)], rows_v)
            pltpu.sync_copy(rows_v, accp.at[idx_v.at[0]], add=True)
            pltpu.sync_copy(ones_v, accc.at[idx_v.at[0]], add=True)
            return _

        lax.fori_loop(0, nch, fn, None)
        plsc.subcore_barrier()

        @pl.when(s == 0)
        def _():
            pltpu.sync_copy(accp, outp_hbm.at[pl.ds(c * G, G)])
            pltpu.sync_copy(accc, outc_hbm.at[pl.ds(c * G, G)])

    return pool_kernel


def _dis_from_cnt(dcnt):
    deg = 1.0 + dcnt[0][:, 0:1] + dcnt[1][:, 0:1]
    return lax.rsqrt(deg)


def _b_body(x_ref, w_ref, dcnt_ref, hs_ref):
    dis = _dis_from_cnt(dcnt_ref)
    z = jnp.dot(x_ref[...], w_ref[...], preferred_element_type=F32)
    hs_ref[...] = z * dis


def _d_body(part_ref, hs_ref, dcnt_ref, w_ref, b_ref, hs2_ref):
    dis = _dis_from_cnt(dcnt_ref)
    q = part_ref[0] + part_ref[1]
    h1 = jnp.maximum(dis * (q + hs_ref[...]) + b_ref[...], 0.0)
    hs2_ref[...] = jnp.dot(h1, w_ref[...], preferred_element_type=F32) * dis


def _e_body(part_ref, hs2_ref, dcnt_ref, b_ref, h2_ref):
    dis = _dis_from_cnt(dcnt_ref)
    q = part_ref[0] + part_ref[1]
    h2_ref[...] = jnp.maximum(dis * (q + hs2_ref[...]) + b_ref[...], 0.0)


def _g_body(p_ref, c_ref, wl_ref, bl_ref, out_ref):
    P = p_ref[0] + p_ref[1]
    cnt = c_ref[0][:, 0:1] + c_ref[1][:, 0:1]
    pooled = P / jnp.maximum(cnt, 1.0)
    out_ref[...] = (
        jnp.dot(pooled, wl_ref[...], preferred_element_type=F32) + bl_ref[...]
    )


def kernel(x, edge_index, batch, W1, b1, W2, b2, Wl, bl):
    N, D = x.shape
    H = W1.shape[1]
    C = Wl.shape[1]
    E = edge_index.shape[1]
    G = 64
    src = edge_index[0]
    dst = edge_index[1]

    R = 2000
    assert N % R == 0
    grid = (N // R,)

    NP = -(-N // (128 * NS)) * (128 * NS)

    dcnt = _make_deg(N, NP, E)(dst).reshape(2, NP, 16)

    hs = pl.pallas_call(
        _b_body,
        grid=grid,
        in_specs=[
            pl.BlockSpec((R, D), lambda i: (i, 0)),
            pl.BlockSpec((D, H), lambda i: (0, 0)),
            pl.BlockSpec((2, R, 16), lambda i: (0, i, 0)),
        ],
        out_specs=pl.BlockSpec((R, H), lambda i: (i, 0)),
        out_shape=jax.ShapeDtypeStruct((N, H), F32),
    )(x, W1, dcnt)

    prop = _make_prop(N, NP, E, H)
    part1 = prop(hs, src, dst).reshape(2, NP, H)

    hs2 = pl.pallas_call(
        _d_body,
        grid=grid,
        in_specs=[
            pl.BlockSpec((2, R, H), lambda i: (0, i, 0)),
            pl.BlockSpec((R, H), lambda i: (i, 0)),
            pl.BlockSpec((2, R, 16), lambda i: (0, i, 0)),
            pl.BlockSpec((H, H), lambda i: (0, 0)),
            pl.BlockSpec((1, H), lambda i: (0, 0)),
        ],
        out_specs=pl.BlockSpec((R, H), lambda i: (i, 0)),
        out_shape=jax.ShapeDtypeStruct((N, H), F32),
    )(part1, hs, dcnt, W2, b1.reshape(1, H))

    part2 = prop(hs2, src, dst).reshape(2, NP, H)

    h2 = pl.pallas_call(
        _e_body,
        grid=grid,
        in_specs=[
            pl.BlockSpec((2, R, H), lambda i: (0, i, 0)),
            pl.BlockSpec((R, H), lambda i: (i, 0)),
            pl.BlockSpec((2, R, 16), lambda i: (0, i, 0)),
            pl.BlockSpec((1, H), lambda i: (0, 0)),
        ],
        out_specs=pl.BlockSpec((R, H), lambda i: (i, 0)),
        out_shape=jax.ShapeDtypeStruct((N, H), F32),
    )(part2, hs2, dcnt, b2.reshape(1, H))

    pooled_p, cnt_p = _make_pool(N, H, G)(h2, batch)
    pooled_p = pooled_p.reshape(2, G, H)
    cnt_p = cnt_p.reshape(2, G, 16)

    out = pl.pallas_call(
        _g_body,
        in_specs=[
            pl.BlockSpec((2, G, H), lambda: (0, 0, 0)),
            pl.BlockSpec((2, G, 16), lambda: (0, 0, 0)),
            pl.BlockSpec((H, C), lambda: (0, 0)),
            pl.BlockSpec((1, C), lambda: (0, 0)),
        ],
        out_specs=pl.BlockSpec((G, C), lambda: (0, 0)),
        out_shape=jax.ShapeDtypeStruct((G, C), F32),
    )(pooled_p, cnt_p, Wl, bl.reshape(1, C))

    return out


# speedup vs baseline: 29.3163x; 2.2675x over previous
import functools

import jax
import jax.numpy as jnp
from jax import lax
from jax.experimental import pallas as pl
from jax.experimental.pallas import tpu as pltpu
from jax.experimental.pallas import tpu_sc as plsc

NC = 2
NS = 16
NW = NC * NS

@functools.cache
def _mesh():
    return plsc.VectorSubcoreMesh(
        core_axis_name="c", subcore_axis_name="s", num_cores=NC, num_subcores=NS
    )

F32 = jnp.float32


def _fill(ref, rows, width, value):
    per_row = width // 16

    def body(j, _):
        r = j // per_row
        k = j % per_row
        ref[r, pl.ds(k * 16, 16)] = jnp.full((16,), value, F32)
        return _

    lax.fori_loop(0, rows * per_row, body, None)


def _make_deg(N, NP, E):
    EPT = E // NW
    K = 80
    assert E % NW == 0 and EPT % K == 0 and NP % (8 * NS) == 0
    NCH = EPT // K
    RPT = NP // NS
    ZR = 128
    assert RPT % ZR == 0

    @functools.partial(
        pl.kernel,
        out_type=jax.ShapeDtypeStruct((2 * NP, 16), F32),
        mesh=_mesh(),
        scratch_types=[
            pltpu.VMEM((EPT,), jnp.int32),
            pltpu.VMEM((K, 16), F32),
            pltpu.VMEM((ZR, 16), F32),
            pltpu.VMEM_SHARED((NP, 16), F32),
        ],
    )
    def deg_kernel(dst_hbm, out_hbm, didx, ones_v, zb_v, acc):
        c = lax.axis_index("c")
        s = lax.axis_index("s")
        base = (c * NS + s) * EPT
        pltpu.sync_copy(dst_hbm.at[pl.ds(base, EPT)], didx)
        _fill(ones_v, K, 16, 1.0)
        _fill(zb_v, ZR, 16, 0.0)

        def fz(j, _):
            pltpu.sync_copy(zb_v, acc.at[pl.ds(s * RPT + j * ZR, ZR)])
            return _

        lax.fori_loop(0, RPT // ZR, fz, None)
        plsc.subcore_barrier()

        def fe(i, _):
            pltpu.sync_copy(ones_v, acc.at[didx.at[pl.ds(i * K, K)]], add=True)
            return _

        lax.fori_loop(0, NCH, fe, None)
        plsc.subcore_barrier()
        pltpu.sync_copy(
            acc.at[pl.ds(s * RPT, RPT)],
            out_hbm.at[pl.ds(c * NP + s * RPT, RPT)],
        )

    return deg_kernel


def _make_prop(N, NP, E, H):
    EPT = E // NW
    K = 80
    assert EPT % K == 0 and NP % (8 * NS) == 0
    NCH = EPT // K
    RPT = NP // NS
    assert RPT % K == 0
    assert NCH % 2 == 1

    @functools.partial(
        pl.kernel,
        out_type=jax.ShapeDtypeStruct((2 * NP, H), F32),
        mesh=_mesh(),
        scratch_types=[
            pltpu.VMEM((EPT,), jnp.int32),
            pltpu.VMEM((EPT,), jnp.int32),
            pltpu.VMEM((K, H), F32),
            pltpu.VMEM((K, H), F32),
            pltpu.VMEM_SHARED((NP, H), F32),
            pltpu.SemaphoreType.DMA,
            pltpu.SemaphoreType.DMA,
        ],
    )
    def prop_kernel(hs_hbm, src_hbm, dst_hbm, out_hbm, sidx, didx,
                    r0, r1, acc, g0, g1):
        rows = [r0, r1]
        sems = [g0, g1]
        c = lax.axis_index("c")
        s = lax.axis_index("s")
        base = (c * NS + s) * EPT
        pltpu.sync_copy(src_hbm.at[pl.ds(base, EPT)], sidx)
        pltpu.sync_copy(dst_hbm.at[pl.ds(base, EPT)], didx)
        _fill(r0, K, H, 0.0)

        def fz(j, _):
            pltpu.sync_copy(r0, acc.at[pl.ds(s * RPT + j * K, K)])
            return _

        lax.fori_loop(0, RPT // K, fz, None)
        plsc.subcore_barrier()

        def g_src(cc):
            return hs_hbm.at[sidx.at[pl.ds(cc * K, K)]]

        pltpu.async_copy(g_src(0), r0, g0)
        pltpu.async_copy(g_src(1), r1, g1)

        def step(j2, _):
            for b in range(2):
                cc = 2 * j2 + b
                pltpu.make_async_copy(g_src(cc), rows[b], sems[b]).wait()
                pltpu.sync_copy(
                    rows[b], acc.at[didx.at[pl.ds(cc * K, K)]], add=True
                )

                @pl.when(cc + 2 < NCH)
                def _():
                    pltpu.async_copy(g_src(cc + 2), rows[b], sems[b])

            return _

        lax.fori_loop(0, NCH // 2, step, None)
        tail = NCH - 1
        pltpu.make_async_copy(g_src(tail), r0, g0).wait()
        pltpu.sync_copy(r0, acc.at[didx.at[pl.ds(tail * K, K)]], add=True)
        plsc.subcore_barrier()
        pltpu.sync_copy(
            acc.at[pl.ds(s * RPT, RPT)],
            out_hbm.at[pl.ds(c * NP + s * RPT, RPT)],
        )

    return prop_kernel


def _make_pool(N, H, G):
    K = 80
    NPT = -(-N // (NW * K)) * K
    assert NPT % K == 0

    @functools.partial(
        pl.kernel,
        out_type=(
            jax.ShapeDtypeStruct((2 * G, H), F32),
            jax.ShapeDtypeStruct((2 * G, 16), F32),
        ),
        mesh=_mesh(),
        scratch_types=[
            pltpu.VMEM((1, K), jnp.int32),
            pltpu.VMEM((K, 16), F32),
            pltpu.VMEM((K, H), F32),
            pltpu.VMEM((G, H), F32),
            pltpu.VMEM((G, 16), F32),
            pltpu.VMEM_SHARED((G, H), F32),
            pltpu.VMEM_SHARED((G, 16), F32),
        ],
    )
    def pool_kernel(h2_hbm, batch_hbm, outp_hbm, outc_hbm, idx_v, ones_v,
                    rows_v, zbp_v, zbc_v, accp, accc):
        c = lax.axis_index("c")
        s = lax.axis_index("s")
        _fill(ones_v, K, 16, 1.0)

        @pl.when(s == 0)
        def _():
            _fill(zbp_v, G, H, 0.0)
            _fill(zbc_v, G, 16, 0.0)
            pltpu.sync_copy(zbp_v, accp)
            pltpu.sync_copy(zbc_v, accc)

        plsc.subcore_barrier()

        wid = c * NS + s
        base = wid * NPT
        todo = jnp.maximum(jnp.minimum(NPT, N - base), 0)
        nch = todo // K

        def fn(j, _):
            b = base + j * K
            pltpu.sync_copy(batch_hbm.at[pl.ds(b, K)], idx_v.at[0])
            pltpu.sync_copy(h2_hbm.at[pl.ds(b, K)], rows_v)
            pltpu.sync_copy(rows_v, accp.at[idx_v.at[0]], add=True)
            pltpu.sync_copy(ones_v, accc.at[idx_v.at[0]], add=True)
            return _

        lax.fori_loop(0, nch, fn, None)
        plsc.subcore_barrier()

        @pl.when(s == 0)
        def _():
            pltpu.sync_copy(accp, outp_hbm.at[pl.ds(c * G, G)])
            pltpu.sync_copy(accc, outc_hbm.at[pl.ds(c * G, G)])

    return pool_kernel


def _dis_from_cnt(dcnt):
    deg = 1.0 + dcnt[0][:, 0:1] + dcnt[1][:, 0:1]
    return lax.rsqrt(deg)


def _b_body(x_ref, w_ref, dcnt_ref, hs_ref):
    dis = _dis_from_cnt(dcnt_ref)
    z = jnp.dot(x_ref[...], w_ref[...], preferred_element_type=F32)
    hs_ref[...] = z * dis


def _d_body(part_ref, hs_ref, dcnt_ref, w_ref, b_ref, hs2_ref):
    dis = _dis_from_cnt(dcnt_ref)
    q = part_ref[0] + part_ref[1]
    h1 = jnp.maximum(dis * (q + hs_ref[...]) + b_ref[...], 0.0)
    hs2_ref[...] = jnp.dot(h1, w_ref[...], preferred_element_type=F32) * dis


def _e_body(part_ref, hs2_ref, dcnt_ref, b_ref, h2_ref):
    dis = _dis_from_cnt(dcnt_ref)
    q = part_ref[0] + part_ref[1]
    h2_ref[...] = jnp.maximum(dis * (q + hs2_ref[...]) + b_ref[...], 0.0)


def _g_body(p_ref, c_ref, wl_ref, bl_ref, out_ref):
    P = p_ref[0] + p_ref[1]
    cnt = c_ref[0][:, 0:1] + c_ref[1][:, 0:1]
    pooled = P / jnp.maximum(cnt, 1.0)
    out_ref[...] = (
        jnp.dot(pooled, wl_ref[...], preferred_element_type=F32) + bl_ref[...]
    )


def kernel(x, edge_index, batch, W1, b1, W2, b2, Wl, bl):
    N, D = x.shape
    H = W1.shape[1]
    C = Wl.shape[1]
    E = edge_index.shape[1]
    G = 64
    src = edge_index[0]
    dst = edge_index[1]

    R = 2000
    assert N % R == 0
    grid = (N // R,)

    NP = -(-N // (128 * NS)) * (128 * NS)

    dcnt = _make_deg(N, NP, E)(dst).reshape(2, NP, 16)

    hs = pl.pallas_call(
        _b_body,
        grid=grid,
        in_specs=[
            pl.BlockSpec((R, D), lambda i: (i, 0)),
            pl.BlockSpec((D, H), lambda i: (0, 0)),
            pl.BlockSpec((2, R, 16), lambda i: (0, i, 0)),
        ],
        out_specs=pl.BlockSpec((R, H), lambda i: (i, 0)),
        out_shape=jax.ShapeDtypeStruct((N, H), F32),
    )(x, W1, dcnt)

    prop = _make_prop(N, NP, E, H)
    part1 = prop(hs, src, dst).reshape(2, NP, H)

    hs2 = pl.pallas_call(
        _d_body,
        grid=grid,
        in_specs=[
            pl.BlockSpec((2, R, H), lambda i: (0, i, 0)),
            pl.BlockSpec((R, H), lambda i: (i, 0)),
            pl.BlockSpec((2, R, 16), lambda i: (0, i, 0)),
            pl.BlockSpec((H, H), lambda i: (0, 0)),
            pl.BlockSpec((1, H), lambda i: (0, 0)),
        ],
        out_specs=pl.BlockSpec((R, H), lambda i: (i, 0)),
        out_shape=jax.ShapeDtypeStruct((N, H), F32),
    )(part1, hs, dcnt, W2, b1.reshape(1, H))

    part2 = prop(hs2, src, dst).reshape(2, NP, H)

    h2 = pl.pallas_call(
        _e_body,
        grid=grid,
        in_specs=[
            pl.BlockSpec((2, R, H), lambda i: (0, i, 0)),
            pl.BlockSpec((R, H), lambda i: (i, 0)),
            pl.BlockSpec((2, R, 16), lambda i: (0, i, 0)),
            pl.BlockSpec((1, H), lambda i: (0, 0)),
        ],
        out_specs=pl.BlockSpec((R, H), lambda i: (i, 0)),
        out_shape=jax.ShapeDtypeStruct((N, H), F32),
    )(part2, hs2, dcnt, b2.reshape(1, H))

    pooled_p, cnt_p = _make_pool(N, H, G)(h2, batch)
    pooled_p = pooled_p.reshape(2, G, H)
    cnt_p = cnt_p.reshape(2, G, 16)

    out = pl.pallas_call(
        _g_body,
        in_specs=[
            pl.BlockSpec((2, G, H), lambda: (0, 0, 0)),
            pl.BlockSpec((2, G, 16), lambda: (0, 0, 0)),
            pl.BlockSpec((H, C), lambda: (0, 0)),
            pl.BlockSpec((1, C), lambda: (0, 0)),
        ],
        out_specs=pl.BlockSpec((G, C), lambda: (0, 0)),
        out_shape=jax.ShapeDtypeStruct((G, C), F32),
    )(pooled_p, cnt_p, Wl, bl.reshape(1, C))

    return out
